# baseline (speedup 1.0000x reference)
import functools

import jax
import jax.numpy as jnp
from jax import lax
from jax.experimental import pallas as pl
from jax.experimental.pallas import tpu as pltpu
from jax.experimental.pallas import tpu_sc as plsc

N = 10000
E = 320000
G = 32
DIN = 128
H = 64
HC = H // 2
NCLS = 10

NC, NS = 2, 16
NW = NC * NS
K = 80
CH = (E // NS) // K
NPAD = 10240
DEG_PER_SUB = NPAD // NS
RPS = N // NS
NBUF = 8

_f32 = jnp.float32
_mesh = plsc.VectorSubcoreMesh(core_axis_name="c", subcore_axis_name="s")
_sc_params = pltpu.CompilerParams(use_tc_tiling_on_sc=False)


def _deg_body(dst_hbm, out_hbm, didx_v, ones_v, zb_v, deg_sh):
    c = lax.axis_index("c")
    s = lax.axis_index("s")
    w = c * NS + s

    def _fill_z(i, _):
        zb_v[pl.ds(i * 16, 16)] = jnp.zeros((16,), _f32)
        return 0

    lax.fori_loop(0, DEG_PER_SUB // 16, _fill_z, 0)

    def _fill_o(i, _):
        ones_v[pl.ds(i * 16, 16)] = jnp.ones((16,), _f32)
        return 0

    lax.fori_loop(0, K // 16, _fill_o, 0)

    pltpu.sync_copy(dst_hbm.at[w], didx_v)
    pltpu.sync_copy(zb_v, deg_sh.at[pl.ds(s * DEG_PER_SUB, DEG_PER_SUB)])
    plsc.subcore_barrier()

    def _scat(i, _):
        pltpu.sync_copy(ones_v, deg_sh.at[didx_v.at[i]], add=True)
        return 0

    lax.fori_loop(0, CH // NC, _scat, 0)
    plsc.subcore_barrier()
    pltpu.sync_copy(deg_sh.at[pl.ds(s * DEG_PER_SUB, DEG_PER_SUB)],
                    out_hbm.at[c, s, 0])


_deg_call = functools.partial(
    pl.kernel,
    out_type=jax.ShapeDtypeStruct((NC, NS, 1, DEG_PER_SUB), _f32),
    mesh=_mesh,
    compiler_params=_sc_params,
    scratch_types=[
        pltpu.VMEM((CH // NC, K), jnp.int32),
        pltpu.VMEM((K,), _f32),
        pltpu.VMEM((DEG_PER_SUB,), _f32),
        pltpu.VMEM_SHARED((NPAD,), _f32),
    ],
)(_deg_body)


CHH = CH // 2
BLK = 125


def _mega_body(g1_hbm, sidx_hbm, didx_hbm, d8_hbm, b1_hbm, out_hbm,
               sidx_v, didx_v, rows, zb_v, accv, g1v, d8v, b1v,
               acc_sh, g2_sh, gsems, ssems):
    c = lax.axis_index("c")
    s = lax.axis_index("s")

    def _fill_z(i, _):
        zb_v[i // (HC // 16), pl.ds((i % (HC // 16)) * 16, 16)] = (
            jnp.zeros((16,), _f32))
        return 0

    lax.fori_loop(0, K * (HC // 16), _fill_z, 0)

    def _zero_acc():
        def _z7(j, _):
            pltpu.sync_copy(zb_v, acc_sh.at[pl.ds(s * RPS + j * K, K)])
            return 0

        lax.fori_loop(0, 7, _z7, 0)
        pltpu.sync_copy(zb_v.at[pl.ds(0, 65)],
                        acc_sh.at[pl.ds(s * RPS + 7 * K, 65)])

    _zero_acc()
    plsc.subcore_barrier()

    def _gather(i, b, table):
        pltpu.async_copy(table.at[sidx_v.at[i]], rows[b], gsems[b])

    def _gwait(i, b, table):
        pltpu.make_async_copy(table.at[sidx_v.at[i]], rows[b], gsems[b]).wait()

    def _scat(i, b):
        pltpu.async_copy(rows[b], acc_sh.at[didx_v.at[i]], ssems[b], add=True)

    def _swait(i, b):
        pltpu.make_async_copy(rows[b], acc_sh.at[didx_v.at[i]], ssems[b]).wait()

    def _edge_pass(table):
        for h in range(2):
            pltpu.sync_copy(sidx_hbm.at[s, pl.ds(h * CHH, CHH)], sidx_v)
            pltpu.sync_copy(didx_hbm.at[s, pl.ds(h * CHH, CHH)], didx_v)
            for b in range(NBUF):
                _gather(b, b, table)

            def _body(k, _):
                i0 = k * NBUF
                for b in range(NBUF):
                    _gwait(i0 + b, b, table)
                    _scat(i0 + b, b)
                for b in range(NBUF):
                    _swait(i0 + b, b)

                    @pl.when(i0 + NBUF + b < CHH)
                    def _():
                        _gather(i0 + NBUF + b, b, table)
                return 0

            nfull = CHH // NBUF
            lax.fori_loop(0, nfull, _body, 0)
            for i in range(nfull * NBUF, CHH):
                b = i % NBUF
                _gwait(i, b, table)
                _scat(i, b)
                _swait(i, b)

    _edge_pass(g1_hbm.at[c])
    plsc.subcore_barrier()

    pltpu.sync_copy(b1_hbm.at[c], b1v)
    b1a = b1v[pl.ds(0, 16)]
    b1b = b1v[pl.ds(16, 16)]
    for bk in range(RPS // BLK):
        r0 = s * RPS + bk * BLK
        pltpu.sync_copy(acc_sh.at[pl.ds(r0, BLK)], accv)
        pltpu.sync_copy(g1_hbm.at[c, pl.ds(r0, BLK)], g1v)
        pltpu.sync_copy(d8_hbm.at[pl.ds(r0, BLK)], d8v)

        def _ew(r, _):
            d = d8v[r, pl.ds(0, 16)]
            va = accv[r, pl.ds(0, 16)] + g1v[r, pl.ds(0, 16)]
            vb = accv[r, pl.ds(16, 16)] + g1v[r, pl.ds(16, 16)]
            ha = jnp.maximum(d * va + b1a, 0.0)
            hb = jnp.maximum(d * vb + b1b, 0.0)
            g1v[r, pl.ds(0, 16)] = d * ha
            g1v[r, pl.ds(16, 16)] = d * hb
            return 0

        lax.fori_loop(0, BLK, _ew, 0)
        pltpu.sync_copy(g1v, g2_sh.at[pl.ds(r0, BLK)])
    _zero_acc()
    plsc.subcore_barrier()

    _edge_pass(g2_sh)
    plsc.subcore_barrier()

    for bk in range(RPS // BLK):
        r0 = s * RPS + bk * BLK
        pltpu.sync_copy(acc_sh.at[pl.ds(r0, BLK)], accv)
        pltpu.sync_copy(g2_sh.at[pl.ds(r0, BLK)], g1v)

        def _add(r, _):
            accv[r, pl.ds(0, 16)] = accv[r, pl.ds(0, 16)] + g1v[r, pl.ds(0, 16)]
            accv[r, pl.ds(16, 16)] = (accv[r, pl.ds(16, 16)]
                                      + g1v[r, pl.ds(16, 16)])
            return 0

        lax.fori_loop(0, BLK, _add, 0)
        pltpu.sync_copy(accv, out_hbm.at[c, s, pl.ds(bk * BLK, BLK)])


_mega_call = functools.partial(
    pl.kernel,
    out_type=jax.ShapeDtypeStruct((NC, NS, RPS, HC), _f32),
    mesh=_mesh,
    compiler_params=_sc_params,
    scratch_types=[
        pltpu.VMEM((CHH, K), jnp.int32),
        pltpu.VMEM((CHH, K), jnp.int32),
        [pltpu.VMEM((K, HC), _f32)] * NBUF,
        pltpu.VMEM((K, HC), _f32),
        pltpu.VMEM((BLK, HC), _f32),
        pltpu.VMEM((BLK, HC), _f32),
        pltpu.VMEM((BLK, 16), _f32),
        pltpu.VMEM((HC,), _f32),
        pltpu.VMEM_SHARED((N, HC), _f32),
        pltpu.VMEM_SHARED((N, HC), _f32),
        [pltpu.SemaphoreType.DMA] * NBUF,
        [pltpu.SemaphoreType.DMA] * NBUF,
    ],
)(_mega_body)


def _tca_body(degp_ref, x_ref, w1_ref, dinv_ref, d8_ref, g1_ref):
    deg = degp_ref[0] + degp_ref[1] + 1.0
    dinv = lax.rsqrt(jnp.maximum(deg, 1.0))
    dinv_ref[...] = dinv
    dn = dinv[:N]
    d8_ref[...] = jnp.broadcast_to(dn, (N, 16))
    p1 = jnp.dot(x_ref[...], w1_ref[...], preferred_element_type=_f32)
    g1 = dn * p1
    g1_ref[0] = g1[:, :HC]
    g1_ref[1] = g1[:, HC:]


def _tca_call(degp, x, w1):
    return pl.pallas_call(
        _tca_body,
        out_shape=[jax.ShapeDtypeStruct((NPAD, 1), _f32),
                   jax.ShapeDtypeStruct((N, 16), _f32),
                   jax.ShapeDtypeStruct((NC, N, HC), _f32)],
    )(degp, x, w1)


def _tcc_body(pre_ref, dinv_ref, w2_ref, b2_ref, batch_ref,
              fc1w_ref, fc1b_ref, fc2w_ref, fc2b_ref, out_ref):
    w2 = w2_ref[...]
    t = (jnp.dot(pre_ref[0], w2[:HC, :], preferred_element_type=_f32)
         + jnp.dot(pre_ref[1], w2[HC:, :], preferred_element_type=_f32))
    h2 = jax.nn.relu(dinv_ref[...][:N] * t + b2_ref[...])
    onehot = (batch_ref[...] ==
              lax.broadcasted_iota(jnp.int32, (N, G), 1)).astype(_f32)
    sums = lax.dot_general(onehot, h2, (((0,), (0,)), ((), ())),
                           preferred_element_type=_f32)
    counts = lax.dot_general(onehot, jnp.ones((N, 1), _f32),
                             (((0,), (0,)), ((), ())),
                             preferred_element_type=_f32)
    pooled = sums / jnp.maximum(counts, 1.0)
    z = jax.nn.relu(jnp.dot(pooled, fc1w_ref[...],
                            preferred_element_type=_f32) + fc1b_ref[...])
    out_ref[...] = jnp.dot(z, fc2w_ref[...],
                           preferred_element_type=_f32) + fc2b_ref[...]


def _tcc_call(pre, dinv, w2, b2, batch, fc1w, fc1b, fc2w, fc2b):
    return pl.pallas_call(
        _tcc_body,
        out_shape=jax.ShapeDtypeStruct((G, NCLS), _f32),
    )(pre, dinv, w2, b2, batch, fc1w, fc1b, fc2w, fc2b)


def kernel(x, edge_index, batch, W1, b1, W2, b2, fc1_w, fc1_b, fc2_w, fc2_b):
    ei = edge_index.astype(jnp.int32)
    src3d = ei[0].reshape(NS, CH, K)
    dst3d = ei[1].reshape(NS, CH, K)
    dst_deg = ei[1].reshape(NW, CH // NC, K)

    degp = _deg_call(dst_deg)
    dinv, d8, g1 = _tca_call(degp.reshape(NC, NPAD, 1), x, W1)
    pre = _mega_call(g1, src3d, dst3d, d8, b1.reshape(NC, HC))
    out = _tcc_call(pre.reshape(NC, N, HC), dinv, W2,
                    b2.reshape(1, 2 * H), batch.reshape(N, 1),
                    fc1_w, fc1_b.reshape(1, H), fc2_w, fc2_b.reshape(1, NCLS))
    return out


# speedup vs baseline: 1.1781x; 1.1781x over previous
import functools

import jax
import jax.numpy as jnp
from jax import lax
from jax.experimental import pallas as pl
from jax.experimental.pallas import tpu as pltpu
from jax.experimental.pallas import tpu_sc as plsc

N = 10000
E = 320000
G = 32
DIN = 128
H = 64
NCLS = 10

NC, NS = 2, 16
NW = NC * NS
K = 80
CH = (E // NW) // K
NPAD = 10240
DEG_PER_SUB = NPAD // NS
ROWS_PER_SUB = N // NS
NBUF = 8

_f32 = jnp.float32
_mesh = plsc.VectorSubcoreMesh(core_axis_name="c", subcore_axis_name="s")
_sc_params = pltpu.CompilerParams(use_tc_tiling_on_sc=False)


def _deg_body(dst_hbm, out_hbm, didx_v, ones_v, zb_v, deg_sh):
    c = lax.axis_index("c")
    s = lax.axis_index("s")
    w = c * NS + s

    def _fill_z(i, _):
        zb_v[pl.ds(i * 16, 16)] = jnp.zeros((16,), _f32)
        return 0

    lax.fori_loop(0, DEG_PER_SUB // 16, _fill_z, 0)

    def _fill_o(i, _):
        ones_v[pl.ds(i * 16, 16)] = jnp.ones((16,), _f32)
        return 0

    lax.fori_loop(0, K // 16, _fill_o, 0)

    pltpu.sync_copy(dst_hbm.at[w], didx_v)
    pltpu.sync_copy(zb_v, deg_sh.at[pl.ds(s * DEG_PER_SUB, DEG_PER_SUB)])
    plsc.subcore_barrier()

    def _scat(i, _):
        pltpu.sync_copy(ones_v, deg_sh.at[didx_v.at[i]], add=True)
        return 0

    lax.fori_loop(0, CH, _scat, 0)
    plsc.subcore_barrier()
    pltpu.sync_copy(deg_sh.at[pl.ds(s * DEG_PER_SUB, DEG_PER_SUB)],
                    out_hbm.at[c, s, 0])


_deg_call = functools.partial(
    pl.kernel,
    out_type=jax.ShapeDtypeStruct((NC, NS, 1, DEG_PER_SUB), _f32),
    mesh=_mesh,
    compiler_params=_sc_params,
    scratch_types=[
        pltpu.VMEM((CH, K), jnp.int32),
        pltpu.VMEM((K,), _f32),
        pltpu.VMEM((DEG_PER_SUB,), _f32),
        pltpu.VMEM_SHARED((NPAD,), _f32),
    ],
)(_deg_body)


def _edge_body(g_hbm, sidx_hbm, didx_hbm, out_hbm,
               sidx_v, didx_v, rows, zb_v, acc_sh, gsems, ssems):
    c = lax.axis_index("c")
    s = lax.axis_index("s")
    w = c * NS + s

    pltpu.sync_copy(sidx_hbm.at[w], sidx_v)
    pltpu.sync_copy(didx_hbm.at[w], didx_v)

    def _fill_z(i, _):
        zb_v[i // (H // 16), pl.ds((i % (H // 16)) * 16, 16)] = jnp.zeros((16,), _f32)
        return 0

    lax.fori_loop(0, 125 * (H // 16), _fill_z, 0)

    def _zero(j, _):
        pltpu.sync_copy(zb_v, acc_sh.at[pl.ds(s * ROWS_PER_SUB + j * 125, 125)])
        return 0

    lax.fori_loop(0, ROWS_PER_SUB // 125, _zero, 0)
    plsc.subcore_barrier()

    def _gather(i, b):
        pltpu.async_copy(g_hbm.at[sidx_v.at[i]], rows[b], gsems[b])

    def _gwait(i, b):
        pltpu.make_async_copy(g_hbm.at[sidx_v.at[i]], rows[b], gsems[b]).wait()

    def _scat(i, b):
        pltpu.async_copy(rows[b], acc_sh.at[didx_v.at[i]], ssems[b], add=True)

    def _swait(i, b):
        pltpu.make_async_copy(rows[b], acc_sh.at[didx_v.at[i]], ssems[b]).wait()

    for b in range(NBUF):
        _gather(b, b)

    def _body(k, _):
        i0 = k * NBUF
        for b in range(NBUF):
            _gwait(i0 + b, b)
            pltpu.sync_copy(rows[b], acc_sh.at[didx_v.at[i0 + b]], add=True)

            @pl.when(i0 + NBUF + b < CH)
            def _():
                _gather(i0 + NBUF + b, b)
        return 0

    nfull = CH // NBUF
    lax.fori_loop(0, nfull, _body, 0)
    for i in range(nfull * NBUF, CH):
        b = i % NBUF
        _gwait(i, b)
        pltpu.sync_copy(rows[b], acc_sh.at[didx_v.at[i]], add=True)

    plsc.subcore_barrier()
    pltpu.sync_copy(acc_sh.at[pl.ds(s * ROWS_PER_SUB, ROWS_PER_SUB)],
                    out_hbm.at[c, s])


_edge_call = functools.partial(
    pl.kernel,
    out_type=jax.ShapeDtypeStruct((NC, NS, ROWS_PER_SUB, H), _f32),
    mesh=_mesh,
    compiler_params=_sc_params,
    scratch_types=[
        pltpu.VMEM((CH, K), jnp.int32),
        pltpu.VMEM((CH, K), jnp.int32),
        [pltpu.VMEM((K, H), _f32)] * NBUF,
        pltpu.VMEM((125, H), _f32),
        pltpu.VMEM_SHARED((N, H), _f32),
        [pltpu.SemaphoreType.DMA] * NBUF,
        [pltpu.SemaphoreType.DMA] * NBUF,
    ],
)(_edge_body)


def _tca_body(degp_ref, x_ref, w1_ref, dinv_ref, g1_ref):
    deg = degp_ref[0] + degp_ref[1] + 1.0
    dinv = lax.rsqrt(jnp.maximum(deg, 1.0))
    dinv_ref[...] = dinv
    p1 = jnp.dot(x_ref[...], w1_ref[...], preferred_element_type=_f32)
    g1_ref[...] = dinv[:N] * p1


def _tca_call(degp, x, w1):
    return pl.pallas_call(
        _tca_body,
        out_shape=[jax.ShapeDtypeStruct((NPAD, 1), _f32),
                   jax.ShapeDtypeStruct((N, H), _f32)],
    )(degp, x, w1)


def _tcb_body(acc_ref, g1_ref, dinv_ref, b1_ref, g1b_ref):
    dinv = dinv_ref[...][:N]
    h = jax.nn.relu(dinv * (acc_ref[0] + acc_ref[1] + g1_ref[...]) + b1_ref[...])
    g1b_ref[...] = dinv * h


def _tcb_call(acc, g1, dinv, b1):
    return pl.pallas_call(
        _tcb_body,
        out_shape=jax.ShapeDtypeStruct((N, H), _f32),
    )(acc, g1, dinv, b1)


def _tcc_body(acc_ref, g1b_ref, dinv_ref, w2_ref, b2_ref, batch_ref,
              fc1w_ref, fc1b_ref, fc2w_ref, fc2b_ref, out_ref):
    t = jnp.dot(acc_ref[0] + acc_ref[1] + g1b_ref[...], w2_ref[...],
                preferred_element_type=_f32)
    h2 = jax.nn.relu(dinv_ref[...][:N] * t + b2_ref[...])
    onehot = (batch_ref[...] ==
              lax.broadcasted_iota(jnp.int32, (N, G), 1)).astype(_f32)
    sums = lax.dot_general(onehot, h2, (((0,), (0,)), ((), ())),
                           preferred_element_type=_f32)
    counts = lax.dot_general(onehot, jnp.ones((N, 1), _f32),
                             (((0,), (0,)), ((), ())),
                             preferred_element_type=_f32)
    pooled = sums / jnp.maximum(counts, 1.0)
    z = jax.nn.relu(jnp.dot(pooled, fc1w_ref[...],
                            preferred_element_type=_f32) + fc1b_ref[...])
    out_ref[...] = jnp.dot(z, fc2w_ref[...],
                           preferred_element_type=_f32) + fc2b_ref[...]


def _tcc_call(acc, g1b, dinv, w2, b2, batch, fc1w, fc1b, fc2w, fc2b):
    return pl.pallas_call(
        _tcc_body,
        out_shape=jax.ShapeDtypeStruct((G, NCLS), _f32),
    )(acc, g1b, dinv, w2, b2, batch, fc1w, fc1b, fc2w, fc2b)


def kernel(x, edge_index, batch, W1, b1, W2, b2, fc1_w, fc1_b, fc2_w, fc2_b):
    ei = edge_index.astype(jnp.int32)
    src3d = ei[0].reshape(NW, CH, K)
    dst3d = ei[1].reshape(NW, CH, K)

    degp = _deg_call(dst3d)
    dinv, g1 = _tca_call(degp.reshape(NC, NPAD, 1), x, W1)
    s1 = _edge_call(g1, src3d, dst3d).reshape(NC, N, H)
    g1b = _tcb_call(s1, g1, dinv, b1.reshape(1, H))
    s2 = _edge_call(g1b, src3d, dst3d).reshape(NC, N, H)
    out = _tcc_call(s2, g1b, dinv, W2, b2.reshape(1, 2 * H),
                    batch.reshape(N, 1), fc1_w, fc1_b.reshape(1, H),
                    fc2_w, fc2_b.reshape(1, NCLS))
    return out


# speedup vs baseline: 1.1785x; 1.0004x over previous
import functools

import jax
import jax.numpy as jnp
from jax import lax
from jax.experimental import pallas as pl
from jax.experimental.pallas import tpu as pltpu
from jax.experimental.pallas import tpu_sc as plsc

N = 10000
E = 320000
G = 32
DIN = 128
H = 64
NCLS = 10

NC, NS = 2, 16
NW = NC * NS
K = 80
CH = (E // NW) // K
NPAD = 10240
DEG_PER_SUB = NPAD // NS
ROWS_PER_SUB = N // NS
NBUF = 8

_f32 = jnp.float32
_mesh = plsc.VectorSubcoreMesh(core_axis_name="c", subcore_axis_name="s")
_sc_params = pltpu.CompilerParams(use_tc_tiling_on_sc=False)


def _deg_body(dst_hbm, out_hbm, didx_v, ones_v, zb_v, deg_sh):
    c = lax.axis_index("c")
    s = lax.axis_index("s")
    w = c * NS + s

    def _fill_z(i, _):
        zb_v[pl.ds(i * 16, 16)] = jnp.zeros((16,), _f32)
        return 0

    lax.fori_loop(0, DEG_PER_SUB // 16, _fill_z, 0)

    def _fill_o(i, _):
        ones_v[pl.ds(i * 16, 16)] = jnp.ones((16,), _f32)
        return 0

    lax.fori_loop(0, K // 16, _fill_o, 0)

    pltpu.sync_copy(dst_hbm.at[w], didx_v)
    pltpu.sync_copy(zb_v, deg_sh.at[pl.ds(s * DEG_PER_SUB, DEG_PER_SUB)])
    plsc.subcore_barrier()

    def _scat(i, _):
        pltpu.sync_copy(ones_v, deg_sh.at[didx_v.at[i]], add=True)
        return 0

    lax.fori_loop(0, CH, _scat, 0)
    plsc.subcore_barrier()
    pltpu.sync_copy(deg_sh.at[pl.ds(s * DEG_PER_SUB, DEG_PER_SUB)],
                    out_hbm.at[c, s, 0])


_deg_call = functools.partial(
    pl.kernel,
    out_type=jax.ShapeDtypeStruct((NC, NS, 1, DEG_PER_SUB), _f32),
    mesh=_mesh,
    compiler_params=_sc_params,
    scratch_types=[
        pltpu.VMEM((CH, K), jnp.int32),
        pltpu.VMEM((K,), _f32),
        pltpu.VMEM((DEG_PER_SUB,), _f32),
        pltpu.VMEM_SHARED((NPAD,), _f32),
    ],
)(_deg_body)


def _edge_body(g_hbm, sidx_hbm, didx_hbm, out_hbm,
               sidx_v, didx_v, rows, zb_v, acc_sh, gsems, ssems):
    c = lax.axis_index("c")
    s = lax.axis_index("s")
    w = c * NS + s

    pltpu.sync_copy(sidx_hbm.at[w], sidx_v)
    pltpu.sync_copy(didx_hbm.at[w], didx_v)

    def _fill_z(i, _):
        zb_v[i // (H // 16), pl.ds((i % (H // 16)) * 16, 16)] = jnp.zeros((16,), _f32)
        return 0

    lax.fori_loop(0, 125 * (H // 16), _fill_z, 0)

    def _zero(j, _):
        pltpu.sync_copy(zb_v, acc_sh.at[pl.ds(s * ROWS_PER_SUB + j * 125, 125)])
        return 0

    lax.fori_loop(0, ROWS_PER_SUB // 125, _zero, 0)
    plsc.subcore_barrier()

    def _gather(i, b):
        pltpu.async_copy(g_hbm.at[sidx_v.at[i]], rows[b], gsems[b])

    def _gwait(i, b):
        pltpu.make_async_copy(g_hbm.at[sidx_v.at[i]], rows[b], gsems[b]).wait()

    def _scat(i, b):
        pltpu.async_copy(rows[b], acc_sh.at[didx_v.at[i]], ssems[b], add=True)

    def _swait(i, b):
        pltpu.make_async_copy(rows[b], acc_sh.at[didx_v.at[i]], ssems[b]).wait()

    for b in range(NBUF):
        _gather(b, b)

    def _body(k, _):
        i0 = k * NBUF
        for b in range(NBUF):
            _gwait(i0 + b, b)
            pltpu.sync_copy(rows[b], acc_sh.at[didx_v.at[i0 + b]], add=True)

            @pl.when(i0 + NBUF + b < CH)
            def _():
                _gather(i0 + NBUF + b, b)
        return 0

    nfull = CH // NBUF
    lax.fori_loop(0, nfull, _body, 0)
    for i in range(nfull * NBUF, CH):
        b = i % NBUF
        _gwait(i, b)
        pltpu.sync_copy(rows[b], acc_sh.at[didx_v.at[i]], add=True)

    plsc.subcore_barrier()
    pltpu.sync_copy(acc_sh.at[pl.ds(s * ROWS_PER_SUB, ROWS_PER_SUB)],
                    out_hbm.at[c, s])


_edge_call = functools.partial(
    pl.kernel,
    out_type=jax.ShapeDtypeStruct((NC, NS, ROWS_PER_SUB, H), _f32),
    mesh=_mesh,
    compiler_params=_sc_params,
    scratch_types=[
        pltpu.VMEM((CH, K), jnp.int32),
        pltpu.VMEM((CH, K), jnp.int32),
        [pltpu.VMEM((K, H), _f32)] * NBUF,
        pltpu.VMEM((125, H), _f32),
        pltpu.VMEM_SHARED((N, H), _f32),
        [pltpu.SemaphoreType.DMA] * NBUF,
        [pltpu.SemaphoreType.DMA] * NBUF,
    ],
)(_edge_body)


def _tc0_body(x_ref, w1_ref, p1_ref):
    p1_ref[...] = jnp.dot(x_ref[...], w1_ref[...], preferred_element_type=_f32)


def _tc0_call(x, w1):
    return pl.pallas_call(
        _tc0_body,
        out_shape=jax.ShapeDtypeStruct((N, H), _f32),
    )(x, w1)


def _tca_body(degp_ref, p1_ref, dinv_ref, g1_ref):
    deg = degp_ref[0] + degp_ref[1] + 1.0
    dinv = lax.rsqrt(jnp.maximum(deg, 1.0))
    dinv_ref[...] = dinv
    g1_ref[...] = dinv[:N] * p1_ref[...]


def _tca_call(degp, p1):
    return pl.pallas_call(
        _tca_body,
        out_shape=[jax.ShapeDtypeStruct((NPAD, 1), _f32),
                   jax.ShapeDtypeStruct((N, H), _f32)],
    )(degp, p1)


def _tcb_body(acc_ref, g1_ref, dinv_ref, b1_ref, g1b_ref):
    dinv = dinv_ref[...][:N]
    h = jax.nn.relu(dinv * (acc_ref[0] + acc_ref[1] + g1_ref[...]) + b1_ref[...])
    g1b_ref[...] = dinv * h


def _tcb_call(acc, g1, dinv, b1):
    return pl.pallas_call(
        _tcb_body,
        out_shape=jax.ShapeDtypeStruct((N, H), _f32),
    )(acc, g1, dinv, b1)


def _tcc_body(acc_ref, g1b_ref, dinv_ref, w2_ref, b2_ref, batch_ref,
              fc1w_ref, fc1b_ref, fc2w_ref, fc2b_ref, out_ref):
    t = jnp.dot(acc_ref[0] + acc_ref[1] + g1b_ref[...], w2_ref[...],
                preferred_element_type=_f32)
    h2 = jax.nn.relu(dinv_ref[...][:N] * t + b2_ref[...])
    onehot = (batch_ref[...] ==
              lax.broadcasted_iota(jnp.int32, (N, G), 1)).astype(_f32)
    sums = lax.dot_general(onehot, h2, (((0,), (0,)), ((), ())),
                           preferred_element_type=_f32)
    counts = lax.dot_general(onehot, jnp.ones((N, 1), _f32),
                             (((0,), (0,)), ((), ())),
                             preferred_element_type=_f32)
    pooled = sums / jnp.maximum(counts, 1.0)
    z = jax.nn.relu(jnp.dot(pooled, fc1w_ref[...],
                            preferred_element_type=_f32) + fc1b_ref[...])
    out_ref[...] = jnp.dot(z, fc2w_ref[...],
                           preferred_element_type=_f32) + fc2b_ref[...]


def _tcc_call(acc, g1b, dinv, w2, b2, batch, fc1w, fc1b, fc2w, fc2b):
    return pl.pallas_call(
        _tcc_body,
        out_shape=jax.ShapeDtypeStruct((G, NCLS), _f32),
    )(acc, g1b, dinv, w2, b2, batch, fc1w, fc1b, fc2w, fc2b)


def kernel(x, edge_index, batch, W1, b1, W2, b2, fc1_w, fc1_b, fc2_w, fc2_b):
    ei = edge_index.astype(jnp.int32)
    src3d = ei[0].reshape(NW, CH, K)
    dst3d = ei[1].reshape(NW, CH, K)

    p1 = _tc0_call(x, W1)
    degp = _deg_call(dst3d)
    dinv, g1 = _tca_call(degp.reshape(NC, NPAD, 1), p1)
    s1 = _edge_call(g1, src3d, dst3d).reshape(NC, N, H)
    g1b = _tcb_call(s1, g1, dinv, b1.reshape(1, H))
    s2 = _edge_call(g1b, src3d, dst3d).reshape(NC, N, H)
    out = _tcc_call(s2, g1b, dinv, W2, b2.reshape(1, 2 * H),
                    batch.reshape(N, 1), fc1_w, fc1_b.reshape(1, H),
                    fc2_w, fc2_b.reshape(1, NCLS))
    return out


# speedup vs baseline: 1.1811x; 1.0022x over previous
import functools

import jax
import jax.numpy as jnp
from jax import lax
from jax.experimental import pallas as pl
from jax.experimental.pallas import tpu as pltpu
from jax.experimental.pallas import tpu_sc as plsc

N = 10000
E = 320000
G = 32
DIN = 128
H = 64
NCLS = 10

NC, NS = 2, 16
NW = NC * NS
K = 80
CH = (E // NW) // K
NPAD = 10240
DEG_PER_SUB = NPAD // NS
ROWS_PER_SUB = N // NS
NBUF = 8

_f32 = jnp.float32
_mesh = plsc.VectorSubcoreMesh(core_axis_name="c", subcore_axis_name="s")
_sc_params = pltpu.CompilerParams(use_tc_tiling_on_sc=False)


def _deg_body(dst_hbm, out_hbm, didx_v, ones_v, zb_v, deg_sh):
    c = lax.axis_index("c")
    s = lax.axis_index("s")
    w = c * NS + s

    def _fill_z(i, _):
        zb_v[pl.ds(i * 16, 16)] = jnp.zeros((16,), _f32)
        return 0

    lax.fori_loop(0, DEG_PER_SUB // 16, _fill_z, 0)

    def _fill_o(i, _):
        ones_v[pl.ds(i * 16, 16)] = jnp.ones((16,), _f32)
        return 0

    lax.fori_loop(0, K // 16, _fill_o, 0)

    pltpu.sync_copy(dst_hbm.at[w], didx_v)
    pltpu.sync_copy(zb_v, deg_sh.at[pl.ds(s * DEG_PER_SUB, DEG_PER_SUB)])
    plsc.subcore_barrier()

    def _scat(i, _):
        pltpu.sync_copy(ones_v, deg_sh.at[didx_v.at[i]], add=True)
        return 0

    lax.fori_loop(0, CH, _scat, 0)
    plsc.subcore_barrier()
    pltpu.sync_copy(deg_sh.at[pl.ds(s * DEG_PER_SUB, DEG_PER_SUB)],
                    out_hbm.at[c, s, 0])


_deg_call = functools.partial(
    pl.kernel,
    out_type=jax.ShapeDtypeStruct((NC, NS, 1, DEG_PER_SUB), _f32),
    mesh=_mesh,
    compiler_params=_sc_params,
    scratch_types=[
        pltpu.VMEM((CH, K), jnp.int32),
        pltpu.VMEM((K,), _f32),
        pltpu.VMEM((DEG_PER_SUB,), _f32),
        pltpu.VMEM_SHARED((NPAD,), _f32),
    ],
)(_deg_body)


def _edge_body(g_hbm, sidx_hbm, didx_hbm, out_hbm,
               sidx_v, didx_v, rows, zb_v, acc_sh, gsems, ssems):
    c = lax.axis_index("c")
    s = lax.axis_index("s")
    w = c * NS + s

    pltpu.sync_copy(sidx_hbm.at[w], sidx_v)
    pltpu.sync_copy(didx_hbm.at[w], didx_v)

    def _fill_z(i, _):
        zb_v[i // (H // 16), pl.ds((i % (H // 16)) * 16, 16)] = jnp.zeros((16,), _f32)
        return 0

    lax.fori_loop(0, 125 * (H // 16), _fill_z, 0)

    def _zero(j, _):
        pltpu.sync_copy(zb_v, acc_sh.at[pl.ds(s * ROWS_PER_SUB + j * 125, 125)])
        return 0

    lax.fori_loop(0, ROWS_PER_SUB // 125, _zero, 0)
    plsc.subcore_barrier()

    def _gather(i, b):
        pltpu.async_copy(g_hbm.at[sidx_v.at[i]], rows[b], gsems[b])

    def _gwait(i, b):
        pltpu.make_async_copy(g_hbm.at[sidx_v.at[i]], rows[b], gsems[b]).wait()

    def _scat(i, b):
        pltpu.async_copy(rows[b], acc_sh.at[didx_v.at[i]], ssems[b], add=True)

    def _swait(i, b):
        pltpu.make_async_copy(rows[b], acc_sh.at[didx_v.at[i]], ssems[b]).wait()

    for b in range(NBUF):
        _gather(b, b)

    def _body(k, _):
        i0 = k * NBUF
        for b in range(NBUF):
            _gwait(i0 + b, b)
            pltpu.sync_copy(rows[b], acc_sh.at[didx_v.at[i0 + b]], add=True)

            @pl.when(i0 + NBUF + b < CH)
            def _():
                _gather(i0 + NBUF + b, b)
        return 0

    nfull = CH // NBUF
    lax.fori_loop(0, nfull, _body, 0)
    for i in range(nfull * NBUF, CH):
        b = i % NBUF
        _gwait(i, b)
        pltpu.sync_copy(rows[b], acc_sh.at[didx_v.at[i]], add=True)

    plsc.subcore_barrier()
    pltpu.sync_copy(acc_sh.at[pl.ds(s * ROWS_PER_SUB, ROWS_PER_SUB)],
                    out_hbm.at[c, s])


_edge_call = functools.partial(
    pl.kernel,
    out_type=jax.ShapeDtypeStruct((NC, NS, ROWS_PER_SUB, H), _f32),
    mesh=_mesh,
    compiler_params=_sc_params,
    scratch_types=[
        pltpu.VMEM((CH, K), jnp.int32),
        pltpu.VMEM((CH, K), jnp.int32),
        [pltpu.VMEM((K, H), _f32)] * NBUF,
        pltpu.VMEM((125, H), _f32),
        pltpu.VMEM_SHARED((N, H), _f32),
        [pltpu.SemaphoreType.DMA] * NBUF,
        [pltpu.SemaphoreType.DMA] * NBUF,
    ],
)(_edge_body)


def _tca_body(degp_ref, x_ref, w1_ref, dinv_ref, g1_ref):
    deg = degp_ref[0] + degp_ref[1] + 1.0
    dinv = lax.rsqrt(jnp.maximum(deg, 1.0))
    dinv_ref[...] = dinv
    p1 = jnp.dot(x_ref[...], w1_ref[...], preferred_element_type=_f32)
    g1_ref[...] = dinv[:N] * p1


def _tca_call(degp, x, w1):
    return pl.pallas_call(
        _tca_body,
        out_shape=[jax.ShapeDtypeStruct((NPAD, 1), _f32),
                   jax.ShapeDtypeStruct((N, H), _f32)],
    )(degp, x, w1)


def _tcb_body(acc_ref, g1_ref, dinv_ref, b1_ref, g1b_ref):
    dinv = dinv_ref[...][:N]
    h = jax.nn.relu(dinv * (acc_ref[0] + acc_ref[1] + g1_ref[...]) + b1_ref[...])
    g1b_ref[...] = dinv * h


def _tcb_call(acc, g1, dinv, b1):
    return pl.pallas_call(
        _tcb_body,
        out_shape=jax.ShapeDtypeStruct((N, H), _f32),
    )(acc, g1, dinv, b1)


def _tcc_body(acc_ref, g1b_ref, dinv_ref, w2_ref, b2_ref, batch_ref,
              fc1w_ref, fc1b_ref, fc2w_ref, fc2b_ref, out_ref):
    t = jnp.dot(acc_ref[0] + acc_ref[1] + g1b_ref[...], w2_ref[...],
                preferred_element_type=_f32)
    h2 = jax.nn.relu(dinv_ref[...][:N] * t + b2_ref[...])
    onehot = (batch_ref[...] ==
              lax.broadcasted_iota(jnp.int32, (N, G), 1)).astype(_f32)
    sums = lax.dot_general(onehot, h2, (((0,), (0,)), ((), ())),
                           preferred_element_type=_f32)
    counts = lax.dot_general(onehot, jnp.ones((N, 1), _f32),
                             (((0,), (0,)), ((), ())),
                             preferred_element_type=_f32)
    pooled = sums / jnp.maximum(counts, 1.0)
    z = jax.nn.relu(jnp.dot(pooled, fc1w_ref[...],
                            preferred_element_type=_f32) + fc1b_ref[...])
    out_ref[...] = jnp.dot(z, fc2w_ref[...],
                           preferred_element_type=_f32) + fc2b_ref[...]


def _tcc_call(acc, g1b, dinv, w2, b2, batch, fc1w, fc1b, fc2w, fc2b):
    return pl.pallas_call(
        _tcc_body,
        out_shape=jax.ShapeDtypeStruct((G, NCLS), _f32),
    )(acc, g1b, dinv, w2, b2, batch, fc1w, fc1b, fc2w, fc2b)


def kernel(x, edge_index, batch, W1, b1, W2, b2, fc1_w, fc1_b, fc2_w, fc2_b):
    ei = edge_index.astype(jnp.int32)
    src3d = ei[0].reshape(NW, CH, K)
    dst3d = ei[1].reshape(NW, CH, K)

    degp = _deg_call(dst3d)
    dinv, g1 = _tca_call(degp.reshape(NC, NPAD, 1), x, W1)
    s1 = _edge_call(g1, src3d, dst3d).reshape(NC, N, H)
    g1b = _tcb_call(s1, g1, dinv, b1.reshape(1, H))
    s2 = _edge_call(g1b, src3d, dst3d).reshape(NC, N, H)
    out = _tcc_call(s2, g1b, dinv, W2, b2.reshape(1, 2 * H),
                    batch.reshape(N, 1), fc1_w, fc1_b.reshape(1, H),
                    fc2_w, fc2_b.reshape(1, NCLS))
    return out
